# Initial kernel scaffold; baseline (speedup 1.0000x reference)
import jax
import jax.numpy as jnp
from jax.experimental import pallas as pl


def kernel(occupation_features, skill_idx, node_x, edge_index, gcn_w1, gcn_b1, gcn_w2, gcn_b2, occ_w1, occ_b1, occ_w2, occ_b2, mp_w1, mp_b1, mp_w2, mp_b2, mp_w3, mp_b3):
    raise NotImplementedError("write your pallas kernel here")


# speedup vs baseline: 13.4030x; 13.4030x over previous
import functools

import jax
import jax.numpy as jnp
from jax import lax
from jax.experimental import pallas as pl
from jax.experimental.pallas import tpu as pltpu
from jax.experimental.pallas import tpu_sc as plsc

N = 10000
E = 160000
D_IN = 256
H = 256
EMB = 128
B = 4096

NC, NS = 2, 16
NW = NC * NS
CHUNK = 128
E_PAD = 163840
PC_W = E_PAD // NW // CHUNK
PC_S = E_PAD // NS // CHUNK
N_ACC = 10112
RZ = N_ACC // NS

_sc_mesh = plsc.VectorSubcoreMesh(core_axis_name="c", subcore_axis_name="s")


@functools.partial(
    pl.kernel,
    out_type=jax.ShapeDtypeStruct((NC, N_ACC, 128), jnp.float32),
    mesh=_sc_mesh,
    scratch_types=[
        pltpu.VMEM((PC_W, CHUNK), jnp.int32),
        pltpu.VMEM((CHUNK, 128), jnp.float32),
        pltpu.VMEM_SHARED((N_ACC, 128), jnp.float32),
        pltpu.SemaphoreType.DMA,
    ],
)
def _deg_kernel(dst_hbm, ones_hbm, zeros_hbm, deg_out, idx_d, ones_v, deg_sh, sem):
    c = lax.axis_index("c")
    s = lax.axis_index("s")
    pltpu.sync_copy(zeros_hbm.at[pl.ds(s * RZ, RZ)], deg_sh.at[pl.ds(s * RZ, RZ)])
    pltpu.sync_copy(ones_hbm, ones_v)
    pltpu.sync_copy(dst_hbm.at[c * NS + s], idx_d)
    plsc.subcore_barrier()

    def body(j, _):
        pltpu.sync_copy(ones_v, deg_sh.at[idx_d.at[j]], add=True)
        return 0

    lax.fori_loop(0, PC_W, body, 0)
    plsc.subcore_barrier()
    pltpu.sync_copy(deg_sh.at[pl.ds(s * RZ, RZ)], deg_out.at[c, pl.ds(s * RZ, RZ)])


@functools.partial(
    pl.kernel,
    out_type=jax.ShapeDtypeStruct((NC, N_ACC, EMB), jnp.float32),
    mesh=_sc_mesh,
    scratch_types=[
        pltpu.VMEM((PC_S, CHUNK), jnp.int32),
        pltpu.VMEM((PC_S, CHUNK), jnp.int32),
        pltpu.VMEM((CHUNK, EMB), jnp.float32),
        pltpu.VMEM_SHARED((N_ACC, EMB), jnp.float32),
        pltpu.SemaphoreType.DMA,
    ],
)
def _msg1_kernel(gtab_hbm, src_hbm, dst_hbm, zeros_hbm, acc_out,
                 idx_s, idx_d, rows_v, acc_sh, sem):
    c = lax.axis_index("c")
    s = lax.axis_index("s")
    pltpu.sync_copy(zeros_hbm.at[pl.ds(s * RZ, RZ)], acc_sh.at[pl.ds(s * RZ, RZ)])
    pltpu.sync_copy(src_hbm.at[c, s], idx_s)
    pltpu.sync_copy(dst_hbm.at[s], idx_d)
    plsc.subcore_barrier()

    def body(j, _):
        pltpu.async_copy(gtab_hbm.at[idx_s.at[j]], rows_v, sem).wait()
        pltpu.sync_copy(rows_v, acc_sh.at[idx_d.at[j]], add=True)
        return 0

    lax.fori_loop(0, PC_S, body, 0)
    plsc.subcore_barrier()
    pltpu.sync_copy(acc_sh.at[pl.ds(s * RZ, RZ)], acc_out.at[c, pl.ds(s * RZ, RZ)])


@functools.partial(
    pl.kernel,
    out_type=jax.ShapeDtypeStruct((NC, N_ACC, EMB), jnp.float32),
    mesh=_sc_mesh,
    scratch_types=[
        pltpu.VMEM((PC_W, CHUNK), jnp.int32),
        pltpu.VMEM((PC_W, CHUNK), jnp.int32),
        pltpu.VMEM((CHUNK, EMB), jnp.float32),
        pltpu.VMEM_SHARED((N_ACC, EMB), jnp.float32),
        pltpu.SemaphoreType.DMA,
    ],
)
def _msg2_kernel(gtab_hbm, src_hbm, dst_hbm, zeros_hbm, acc_out,
                 idx_s, idx_d, rows_v, acc_sh, sem):
    c = lax.axis_index("c")
    s = lax.axis_index("s")
    w = c * NS + s
    pltpu.sync_copy(zeros_hbm.at[pl.ds(s * RZ, RZ)], acc_sh.at[pl.ds(s * RZ, RZ)])
    pltpu.sync_copy(src_hbm.at[w], idx_s)
    pltpu.sync_copy(dst_hbm.at[w], idx_d)
    plsc.subcore_barrier()

    def body(j, _):
        pltpu.async_copy(gtab_hbm.at[idx_s.at[j]], rows_v, sem).wait()
        pltpu.sync_copy(rows_v, acc_sh.at[idx_d.at[j]], add=True)
        return 0

    lax.fori_loop(0, PC_W, body, 0)
    plsc.subcore_barrier()
    pltpu.sync_copy(acc_sh.at[pl.ds(s * RZ, RZ)], acc_out.at[c, pl.ds(s * RZ, RZ)])


_BG = B // NW


@functools.partial(
    pl.kernel,
    out_type=jax.ShapeDtypeStruct((B, EMB), jnp.float32),
    mesh=_sc_mesh,
    scratch_types=[
        pltpu.VMEM((_BG,), jnp.int32),
        pltpu.VMEM((_BG, EMB), jnp.float32),
        pltpu.SemaphoreType.DMA,
    ],
)
def _gather_kernel(emb_hbm, idx_hbm, out_hbm, idx_v, rows_v, sem):
    c = lax.axis_index("c")
    s = lax.axis_index("s")
    w = c * NS + s
    pltpu.sync_copy(idx_hbm.at[pl.ds(w * _BG, _BG)], idx_v)
    pltpu.async_copy(emb_hbm.at[idx_v], rows_v, sem).wait()
    pltpu.sync_copy(rows_v, out_hbm.at[pl.ds(w * _BG, _BG)])


_RB = 400


def _dis_from_degs(degs_ref):
    deg = degs_ref[0, :, 0:1] + degs_ref[1, :, 0:1] + 1.0
    return lax.rsqrt(deg)


def _tc_l1_body(x_ref, w1_ref, degs_ref, out_ref):
    dis = _dis_from_degs(degs_ref)
    h = jnp.dot(x_ref[...], w1_ref[...], preferred_element_type=jnp.float32)
    g = h * dis
    out_ref[0] = g[:, :EMB]
    out_ref[1] = g[:, EMB:]


def _tc_l2_body(acc_ref, g_ref, degs_ref, b1_ref, w2_ref, out_ref):
    dis = _dis_from_degs(degs_ref)
    o0 = jnp.maximum(dis * (acc_ref[0] + g_ref[0]) + b1_ref[0, :EMB], 0.0)
    o1 = jnp.maximum(dis * (acc_ref[1] + g_ref[1]) + b1_ref[0, EMB:], 0.0)
    h2 = (jnp.dot(o0, w2_ref[:EMB, :], preferred_element_type=jnp.float32)
          + jnp.dot(o1, w2_ref[EMB:, :], preferred_element_type=jnp.float32))
    out_ref[...] = h2 * dis


def _tc_emb_body(acc_ref, g_ref, degs_ref, b2_ref, out_ref):
    dis = _dis_from_degs(degs_ref)
    out_ref[...] = dis * (acc_ref[0] + acc_ref[1] + g_ref[...]) + b2_ref[...]


_MB = 512


def _tc_mlp_body(x_ref, sk_ref, ow1_ref, ob1_ref, ow2_ref, ob2_ref,
                 mw1_ref, mb1_ref, mw2_ref, mb2_ref, mw3_ref, mb3_ref, out_ref):
    o = jnp.maximum(
        jnp.dot(x_ref[...], ow1_ref[...], preferred_element_type=jnp.float32)
        + ob1_ref[...], 0.0)
    occ = jnp.dot(o, ow2_ref[...], preferred_element_type=jnp.float32) + ob2_ref[...]
    z = (jnp.dot(occ, mw1_ref[:EMB, :], preferred_element_type=jnp.float32)
         + jnp.dot(sk_ref[...], mw1_ref[EMB:, :], preferred_element_type=jnp.float32)
         + mb1_ref[...])
    z = jnp.maximum(z, 0.0)
    z = jnp.maximum(
        jnp.dot(z, mw2_ref[...], preferred_element_type=jnp.float32) + mb2_ref[...],
        0.0)
    out_ref[...] = jax.nn.sigmoid(
        jnp.dot(z, mw3_ref[...], preferred_element_type=jnp.float32) + mb3_ref[...])


def _row_block(*dims):
    shape = dims
    nlead = len(dims) - 2

    def imap(i):
        return (0,) * nlead + (i,) + (0,)

    return pl.BlockSpec(shape, imap)


def _full(shape):
    return pl.BlockSpec(shape, lambda i: (0,) * len(shape))


def kernel(occupation_features, skill_idx, node_x, edge_index,
           gcn_w1, gcn_b1, gcn_w2, gcn_b2,
           occ_w1, occ_b1, occ_w2, occ_b2,
           mp_w1, mp_b1, mp_w2, mp_b2, mp_w3, mp_b3):
    src = edge_index[0].astype(jnp.int32)
    dst = edge_index[1].astype(jnp.int32)
    pad = E_PAD - E
    ar = jnp.arange(pad, dtype=jnp.int32)
    src_p = jnp.concatenate([src, (ar * 997) % N])
    dst_p = jnp.concatenate([dst, N + (ar % 16)])

    src3w = src_p.reshape(NW, PC_W, CHUNK)
    dst3w = dst_p.reshape(NW, PC_W, CHUNK)
    src3s = src_p.reshape(NS, PC_S, CHUNK)
    dst3s = dst_p.reshape(NS, PC_S, CHUNK)
    srcall = jnp.stack([src3s, src3s + N])

    ones128 = jnp.ones((CHUNK, 128), jnp.float32)
    zeros128 = jnp.zeros((N_ACC, EMB), jnp.float32)

    degs = _deg_kernel(dst3w, ones128, zeros128)

    gtab1 = pl.pallas_call(
        _tc_l1_body,
        grid=(N // _RB,),
        in_specs=[_row_block(_RB, D_IN), _full((D_IN, H)), _row_block(2, _RB, 128)],
        out_specs=_row_block(2, _RB, EMB),
        out_shape=jax.ShapeDtypeStruct((2, N, EMB), jnp.float32),
    )(node_x, gcn_w1, degs)

    acc1 = _msg1_kernel(gtab1.reshape(2 * N, EMB), srcall, dst3s, zeros128)

    g2 = pl.pallas_call(
        _tc_l2_body,
        grid=(N // _RB,),
        in_specs=[_row_block(2, _RB, EMB), _row_block(2, _RB, EMB),
                  _row_block(2, _RB, 128), _full((1, H)), _full((H, EMB))],
        out_specs=_row_block(_RB, EMB),
        out_shape=jax.ShapeDtypeStruct((N, EMB), jnp.float32),
    )(acc1, gtab1, degs, gcn_b1.reshape(1, H), gcn_w2)

    acc2 = _msg2_kernel(g2, src3w, dst3w, zeros128)

    emb = pl.pallas_call(
        _tc_emb_body,
        grid=(N // _RB,),
        in_specs=[_row_block(2, _RB, EMB), _row_block(_RB, EMB),
                  _row_block(2, _RB, 128), _full((1, EMB))],
        out_specs=_row_block(_RB, EMB),
        out_shape=jax.ShapeDtypeStruct((N, EMB), jnp.float32),
    )(acc2, g2, degs, gcn_b2.reshape(1, EMB))

    skill_emb = _gather_kernel(emb, skill_idx.astype(jnp.int32))

    scores = pl.pallas_call(
        _tc_mlp_body,
        grid=(B // _MB,),
        in_specs=[_row_block(_MB, D_IN), _row_block(_MB, EMB),
                  _full((D_IN, H)), _full((1, H)), _full((H, EMB)), _full((1, EMB)),
                  _full((2 * EMB, H)), _full((1, H)), _full((H, H // 2)),
                  _full((1, H // 2)), _full((H // 2, 1)), _full((1, 1))],
        out_specs=_row_block(_MB, 1),
        out_shape=jax.ShapeDtypeStruct((B, 1), jnp.float32),
    )(occupation_features, skill_emb,
      occ_w1, occ_b1.reshape(1, H), occ_w2, occ_b2.reshape(1, EMB),
      mp_w1, mp_b1.reshape(1, H), mp_w2, mp_b2.reshape(1, H // 2),
      mp_w3, mp_b3.reshape(1, 1))

    return scores.reshape(B)


# speedup vs baseline: 16.8421x; 1.2566x over previous
import functools

import jax
import jax.numpy as jnp
from jax import lax
from jax.experimental import pallas as pl
from jax.experimental.pallas import tpu as pltpu
from jax.experimental.pallas import tpu_sc as plsc

N = 10000
E = 160000
D_IN = 256
H = 256
EMB = 128
B = 4096

NC, NS = 2, 16
NW = NC * NS
CHUNK = 96
E_PAD = 165888
PC_W = E_PAD // NW // CHUNK
PC_S = E_PAD // NS // CHUNK
HALF = PC_S // 2
N_ACC = 10112
RZ = N_ACC // NS

_sc_mesh = plsc.VectorSubcoreMesh(core_axis_name="c", subcore_axis_name="s")


@functools.partial(
    pl.kernel,
    out_type=jax.ShapeDtypeStruct((NC, N_ACC, 128), jnp.float32),
    name="sc_deg",
    mesh=_sc_mesh,
    scratch_types=[
        pltpu.VMEM((PC_W, CHUNK), jnp.int32),
        pltpu.VMEM((CHUNK, 128), jnp.float32),
        pltpu.VMEM_SHARED((N_ACC, 128), jnp.float32),
        pltpu.SemaphoreType.DMA,
    ],
)
def _deg_kernel(dst_hbm, ones_hbm, zeros_hbm, deg_out, idx_d, ones_v, deg_sh, sem):
    c = lax.axis_index("c")
    s = lax.axis_index("s")
    pltpu.sync_copy(zeros_hbm.at[pl.ds(s * RZ, RZ)], deg_sh.at[pl.ds(s * RZ, RZ)])
    pltpu.sync_copy(ones_hbm, ones_v)
    pltpu.sync_copy(dst_hbm.at[c * NS + s], idx_d)
    plsc.subcore_barrier()

    def body(j, _):
        pltpu.sync_copy(ones_v, deg_sh.at[idx_d.at[j]], add=True)
        return 0

    lax.fori_loop(0, PC_W, body, 0)
    plsc.subcore_barrier()
    pltpu.sync_copy(deg_sh.at[pl.ds(s * RZ, RZ)], deg_out.at[c, pl.ds(s * RZ, RZ)])


def _pipelined_msg(gtab_hbm, idx_s, idx_d, rows0, rows1, acc_sh, sem0, sem1,
                   nchunks):
    pltpu.async_copy(gtab_hbm.at[idx_s.at[0]], rows0, sem0)

    def body(jj, _):
        j0 = 2 * jj
        j1 = j0 + 1
        pltpu.async_copy(gtab_hbm.at[idx_s.at[j1]], rows1, sem1)
        pltpu.make_async_copy(gtab_hbm.at[idx_s.at[j0]], rows0, sem0).wait()
        pltpu.sync_copy(rows0, acc_sh.at[idx_d.at[j0]], add=True)

        @pl.when(j0 + 2 < nchunks)
        def _():
            pltpu.async_copy(gtab_hbm.at[idx_s.at[j0 + 2]], rows0, sem0)

        pltpu.make_async_copy(gtab_hbm.at[idx_s.at[j1]], rows1, sem1).wait()
        pltpu.sync_copy(rows1, acc_sh.at[idx_d.at[j1]], add=True)
        return 0

    lax.fori_loop(0, nchunks // 2, body, 0)


@functools.partial(
    pl.kernel,
    out_type=jax.ShapeDtypeStruct((NC, N_ACC, EMB), jnp.float32),
    name="sc_msg1",
    mesh=_sc_mesh,
    scratch_types=[
        pltpu.VMEM((HALF, CHUNK), jnp.int32),
        pltpu.VMEM((HALF, CHUNK), jnp.int32),
        pltpu.VMEM((CHUNK, EMB), jnp.float32),
        pltpu.VMEM((CHUNK, EMB), jnp.float32),
        pltpu.VMEM_SHARED((N_ACC, EMB), jnp.float32),
        pltpu.SemaphoreType.DMA,
        pltpu.SemaphoreType.DMA,
    ],
)
def _msg1_kernel(gtab_hbm, src_hbm, dst_hbm, zeros_hbm, acc_out,
                 idx_s, idx_d, rows0, rows1, acc_sh, sem0, sem1):
    c = lax.axis_index("c")
    s = lax.axis_index("s")
    pltpu.sync_copy(zeros_hbm.at[pl.ds(s * RZ, RZ)], acc_sh.at[pl.ds(s * RZ, RZ)])
    plsc.subcore_barrier()
    for ph in range(2):
        pltpu.sync_copy(src_hbm.at[c, s, ph], idx_s)
        pltpu.sync_copy(dst_hbm.at[s, ph], idx_d)
        _pipelined_msg(gtab_hbm, idx_s, idx_d, rows0, rows1, acc_sh, sem0, sem1,
                       HALF)
    plsc.subcore_barrier()
    pltpu.sync_copy(acc_sh.at[pl.ds(s * RZ, RZ)], acc_out.at[c, pl.ds(s * RZ, RZ)])


@functools.partial(
    pl.kernel,
    out_type=jax.ShapeDtypeStruct((NC, N_ACC, EMB), jnp.float32),
    name="sc_msg2",
    mesh=_sc_mesh,
    scratch_types=[
        pltpu.VMEM((PC_W, CHUNK), jnp.int32),
        pltpu.VMEM((PC_W, CHUNK), jnp.int32),
        pltpu.VMEM((CHUNK, EMB), jnp.float32),
        pltpu.VMEM((CHUNK, EMB), jnp.float32),
        pltpu.VMEM_SHARED((N_ACC, EMB), jnp.float32),
        pltpu.SemaphoreType.DMA,
        pltpu.SemaphoreType.DMA,
    ],
)
def _msg2_kernel(gtab_hbm, src_hbm, dst_hbm, zeros_hbm, acc_out,
                 idx_s, idx_d, rows0, rows1, acc_sh, sem0, sem1):
    c = lax.axis_index("c")
    s = lax.axis_index("s")
    w = c * NS + s
    pltpu.sync_copy(zeros_hbm.at[pl.ds(s * RZ, RZ)], acc_sh.at[pl.ds(s * RZ, RZ)])
    pltpu.sync_copy(src_hbm.at[w], idx_s)
    pltpu.sync_copy(dst_hbm.at[w], idx_d)
    plsc.subcore_barrier()
    _pipelined_msg(gtab_hbm, idx_s, idx_d, rows0, rows1, acc_sh, sem0, sem1, PC_W)
    plsc.subcore_barrier()
    pltpu.sync_copy(acc_sh.at[pl.ds(s * RZ, RZ)], acc_out.at[c, pl.ds(s * RZ, RZ)])


_BG = B // NW


@functools.partial(
    pl.kernel,
    out_type=jax.ShapeDtypeStruct((B, EMB), jnp.float32),
    name="sc_gather",
    mesh=_sc_mesh,
    scratch_types=[
        pltpu.VMEM((_BG,), jnp.int32),
        pltpu.VMEM((_BG, EMB), jnp.float32),
        pltpu.SemaphoreType.DMA,
    ],
)
def _gather_kernel(emb_hbm, idx_hbm, out_hbm, idx_v, rows_v, sem):
    c = lax.axis_index("c")
    s = lax.axis_index("s")
    w = c * NS + s
    pltpu.sync_copy(idx_hbm.at[pl.ds(w * _BG, _BG)], idx_v)
    pltpu.async_copy(emb_hbm.at[idx_v], rows_v, sem).wait()
    pltpu.sync_copy(rows_v, out_hbm.at[pl.ds(w * _BG, _BG)])


_RB = 400


def _dis_from_degs(degs_ref):
    deg = degs_ref[0, :, 0:1] + degs_ref[1, :, 0:1] + 1.0
    return lax.rsqrt(deg)


def _tc_l1_body(x_ref, w1_ref, degs_ref, out_ref):
    dis = _dis_from_degs(degs_ref)
    h = jnp.dot(x_ref[...], w1_ref[...], preferred_element_type=jnp.float32)
    g = h * dis
    out_ref[0] = g[:, :EMB]
    out_ref[1] = g[:, EMB:]


def _tc_l2_body(acc_ref, g_ref, degs_ref, b1_ref, w2_ref, out_ref):
    dis = _dis_from_degs(degs_ref)
    o0 = jnp.maximum(dis * (acc_ref[0] + g_ref[0]) + b1_ref[0, :EMB], 0.0)
    o1 = jnp.maximum(dis * (acc_ref[1] + g_ref[1]) + b1_ref[0, EMB:], 0.0)
    h2 = (jnp.dot(o0, w2_ref[:EMB, :], preferred_element_type=jnp.float32)
          + jnp.dot(o1, w2_ref[EMB:, :], preferred_element_type=jnp.float32))
    out_ref[...] = h2 * dis


def _tc_emb_body(acc_ref, g_ref, degs_ref, b2_ref, out_ref):
    dis = _dis_from_degs(degs_ref)
    out_ref[...] = dis * (acc_ref[0] + acc_ref[1] + g_ref[...]) + b2_ref[...]


_MB = 512


def _tc_mlp_body(x_ref, sk_ref, ow1_ref, ob1_ref, ow2_ref, ob2_ref,
                 mw1_ref, mb1_ref, mw2_ref, mb2_ref, mw3_ref, mb3_ref, out_ref):
    o = jnp.maximum(
        jnp.dot(x_ref[...], ow1_ref[...], preferred_element_type=jnp.float32)
        + ob1_ref[...], 0.0)
    occ = jnp.dot(o, ow2_ref[...], preferred_element_type=jnp.float32) + ob2_ref[...]
    z = (jnp.dot(occ, mw1_ref[:EMB, :], preferred_element_type=jnp.float32)
         + jnp.dot(sk_ref[...], mw1_ref[EMB:, :], preferred_element_type=jnp.float32)
         + mb1_ref[...])
    z = jnp.maximum(z, 0.0)
    z = jnp.maximum(
        jnp.dot(z, mw2_ref[...], preferred_element_type=jnp.float32) + mb2_ref[...],
        0.0)
    out_ref[...] = jax.nn.sigmoid(
        jnp.dot(z, mw3_ref[...], preferred_element_type=jnp.float32) + mb3_ref[...])


def _row_block(*dims):
    shape = dims
    nlead = len(dims) - 2

    def imap(i):
        return (0,) * nlead + (i,) + (0,)

    return pl.BlockSpec(shape, imap)


def _full(shape):
    return pl.BlockSpec(shape, lambda i: (0,) * len(shape))


def kernel(occupation_features, skill_idx, node_x, edge_index,
           gcn_w1, gcn_b1, gcn_w2, gcn_b2,
           occ_w1, occ_b1, occ_w2, occ_b2,
           mp_w1, mp_b1, mp_w2, mp_b2, mp_w3, mp_b3):
    src = edge_index[0].astype(jnp.int32)
    dst = edge_index[1].astype(jnp.int32)
    pad = E_PAD - E
    ar = jnp.arange(pad, dtype=jnp.int32)
    src_p = jnp.concatenate([src, (ar * 997) % N])
    dst_p = jnp.concatenate([dst, N + (ar % 16)])

    src3w = src_p.reshape(NW, PC_W, CHUNK)
    dst3w = dst_p.reshape(NW, PC_W, CHUNK)
    src3s = src_p.reshape(NS, 2, HALF, CHUNK)
    dst3s = dst_p.reshape(NS, 2, HALF, CHUNK)
    srcall = jnp.stack([src3s, src3s + N])

    ones128 = jnp.ones((CHUNK, 128), jnp.float32)
    zeros128 = jnp.zeros((N_ACC, EMB), jnp.float32)

    degs = _deg_kernel(dst3w, ones128, zeros128)

    gtab1 = pl.pallas_call(
        _tc_l1_body,
        grid=(N // _RB,),
        in_specs=[_row_block(_RB, D_IN), _full((D_IN, H)), _row_block(2, _RB, 128)],
        out_specs=_row_block(2, _RB, EMB),
        out_shape=jax.ShapeDtypeStruct((2, N, EMB), jnp.float32),
    )(node_x, gcn_w1, degs)

    acc1 = _msg1_kernel(gtab1.reshape(2 * N, EMB), srcall, dst3s, zeros128)

    g2 = pl.pallas_call(
        _tc_l2_body,
        grid=(N // _RB,),
        in_specs=[_row_block(2, _RB, EMB), _row_block(2, _RB, EMB),
                  _row_block(2, _RB, 128), _full((1, H)), _full((H, EMB))],
        out_specs=_row_block(_RB, EMB),
        out_shape=jax.ShapeDtypeStruct((N, EMB), jnp.float32),
    )(acc1, gtab1, degs, gcn_b1.reshape(1, H), gcn_w2)

    acc2 = _msg2_kernel(g2, src3w, dst3w, zeros128)

    emb = pl.pallas_call(
        _tc_emb_body,
        grid=(N // _RB,),
        in_specs=[_row_block(2, _RB, EMB), _row_block(_RB, EMB),
                  _row_block(2, _RB, 128), _full((1, EMB))],
        out_specs=_row_block(_RB, EMB),
        out_shape=jax.ShapeDtypeStruct((N, EMB), jnp.float32),
    )(acc2, g2, degs, gcn_b2.reshape(1, EMB))

    skill_emb = _gather_kernel(emb, skill_idx.astype(jnp.int32))

    scores = pl.pallas_call(
        _tc_mlp_body,
        grid=(B // _MB,),
        in_specs=[_row_block(_MB, D_IN), _row_block(_MB, EMB),
                  _full((D_IN, H)), _full((1, H)), _full((H, EMB)), _full((1, EMB)),
                  _full((2 * EMB, H)), _full((1, H)), _full((H, H // 2)),
                  _full((1, H // 2)), _full((H // 2, 1)), _full((1, 1))],
        out_specs=_row_block(_MB, 1),
        out_shape=jax.ShapeDtypeStruct((B, 1), jnp.float32),
    )(occupation_features, skill_emb,
      occ_w1, occ_b1.reshape(1, H), occ_w2, occ_b2.reshape(1, EMB),
      mp_w1, mp_b1.reshape(1, H), mp_w2, mp_b2.reshape(1, H // 2),
      mp_w3, mp_b3.reshape(1, 1))

    return scores.reshape(B)


# speedup vs baseline: 16.9122x; 1.0042x over previous
import functools

import jax
import jax.numpy as jnp
from jax import lax
from jax.experimental import pallas as pl
from jax.experimental.pallas import tpu as pltpu
from jax.experimental.pallas import tpu_sc as plsc

N = 10000
E = 160000
D_IN = 256
H = 256
EMB = 128
B = 4096

NC, NS = 2, 16
NW = NC * NS
CHUNK = 96
E_PAD = 165888
PC_W = E_PAD // NW // CHUNK
PC_S = E_PAD // NS // CHUNK
HALF = PC_S // 2
N_ACC = 10112
RZ = N_ACC // NS

_sc_mesh = plsc.VectorSubcoreMesh(core_axis_name="c", subcore_axis_name="s")


@functools.partial(
    pl.kernel,
    out_type=jax.ShapeDtypeStruct((NC, N_ACC, 128), jnp.float32),
    name="sc_deg",
    mesh=_sc_mesh,
    scratch_types=[
        pltpu.VMEM((PC_W, CHUNK), jnp.int32),
        pltpu.VMEM((CHUNK, 128), jnp.float32),
        pltpu.VMEM_SHARED((N_ACC, 128), jnp.float32),
        pltpu.SemaphoreType.DMA,
    ],
)
def _deg_kernel(dst_hbm, ones_hbm, zeros_hbm, deg_out, idx_d, ones_v, deg_sh, sem):
    c = lax.axis_index("c")
    s = lax.axis_index("s")
    pltpu.sync_copy(zeros_hbm.at[pl.ds(s * RZ, RZ)], deg_sh.at[pl.ds(s * RZ, RZ)])
    pltpu.sync_copy(ones_hbm, ones_v)
    pltpu.sync_copy(dst_hbm.at[c * NS + s], idx_d)
    plsc.subcore_barrier()

    def body(j, _):
        pltpu.sync_copy(ones_v, deg_sh.at[idx_d.at[j]], add=True)
        return 0

    lax.fori_loop(0, PC_W, body, 0)
    plsc.subcore_barrier()
    pltpu.sync_copy(deg_sh.at[pl.ds(s * RZ, RZ)], deg_out.at[c, pl.ds(s * RZ, RZ)])


def _pipelined_msg(gtab_hbm, idx_s, idx_d, rows0, rows1, acc_sh, sem0, sem1,
                   nchunks):
    pltpu.async_copy(gtab_hbm.at[idx_s.at[0]], rows0, sem0)

    def body(jj, _):
        j0 = 2 * jj
        j1 = j0 + 1
        pltpu.async_copy(gtab_hbm.at[idx_s.at[j1]], rows1, sem1)
        pltpu.make_async_copy(gtab_hbm.at[idx_s.at[j0]], rows0, sem0).wait()
        pltpu.sync_copy(rows0, acc_sh.at[idx_d.at[j0]], add=True)

        @pl.when(j0 + 2 < nchunks)
        def _():
            pltpu.async_copy(gtab_hbm.at[idx_s.at[j0 + 2]], rows0, sem0)

        pltpu.make_async_copy(gtab_hbm.at[idx_s.at[j1]], rows1, sem1).wait()
        pltpu.sync_copy(rows1, acc_sh.at[idx_d.at[j1]], add=True)
        return 0

    lax.fori_loop(0, nchunks // 2, body, 0)


@functools.partial(
    pl.kernel,
    out_type=jax.ShapeDtypeStruct((NC, N_ACC, EMB), jnp.float32),
    name="sc_msg1",
    mesh=_sc_mesh,
    scratch_types=[
        pltpu.VMEM((HALF, CHUNK), jnp.int32),
        pltpu.VMEM((HALF, CHUNK), jnp.int32),
        pltpu.VMEM((CHUNK, EMB), jnp.float32),
        pltpu.VMEM((CHUNK, EMB), jnp.float32),
        pltpu.VMEM_SHARED((N_ACC, EMB), jnp.float32),
        pltpu.SemaphoreType.DMA,
        pltpu.SemaphoreType.DMA,
    ],
)
def _msg1_kernel(gtab_hbm, src_hbm, dst_hbm, zeros_hbm, acc_out,
                 idx_s, idx_d, rows0, rows1, acc_sh, sem0, sem1):
    c = lax.axis_index("c")
    s = lax.axis_index("s")
    pltpu.sync_copy(zeros_hbm.at[pl.ds(s * RZ, RZ)], acc_sh.at[pl.ds(s * RZ, RZ)])
    plsc.subcore_barrier()
    for ph in range(2):
        pltpu.sync_copy(src_hbm.at[c, s, ph], idx_s)
        pltpu.sync_copy(dst_hbm.at[s, ph], idx_d)
        _pipelined_msg(gtab_hbm, idx_s, idx_d, rows0, rows1, acc_sh, sem0, sem1,
                       HALF)
    plsc.subcore_barrier()
    pltpu.sync_copy(acc_sh.at[pl.ds(s * RZ, RZ)], acc_out.at[c, pl.ds(s * RZ, RZ)])


@functools.partial(
    pl.kernel,
    out_type=jax.ShapeDtypeStruct((NC, N_ACC, EMB), jnp.float32),
    name="sc_msg2",
    mesh=_sc_mesh,
    scratch_types=[
        pltpu.VMEM((PC_W, CHUNK), jnp.int32),
        pltpu.VMEM((PC_W, CHUNK), jnp.int32),
        pltpu.VMEM((CHUNK, EMB), jnp.float32),
        pltpu.VMEM((CHUNK, EMB), jnp.float32),
        pltpu.VMEM_SHARED((N_ACC, EMB), jnp.float32),
        pltpu.SemaphoreType.DMA,
        pltpu.SemaphoreType.DMA,
    ],
)
def _msg2_kernel(gtab_hbm, src_hbm, dst_hbm, zeros_hbm, acc_out,
                 idx_s, idx_d, rows0, rows1, acc_sh, sem0, sem1):
    c = lax.axis_index("c")
    s = lax.axis_index("s")
    w = c * NS + s
    pltpu.sync_copy(zeros_hbm.at[pl.ds(s * RZ, RZ)], acc_sh.at[pl.ds(s * RZ, RZ)])
    pltpu.sync_copy(src_hbm.at[w], idx_s)
    pltpu.sync_copy(dst_hbm.at[w], idx_d)
    plsc.subcore_barrier()
    _pipelined_msg(gtab_hbm, idx_s, idx_d, rows0, rows1, acc_sh, sem0, sem1, PC_W)
    plsc.subcore_barrier()
    pltpu.sync_copy(acc_sh.at[pl.ds(s * RZ, RZ)], acc_out.at[c, pl.ds(s * RZ, RZ)])


_BG = B // NW


@functools.partial(
    pl.kernel,
    out_type=jax.ShapeDtypeStruct((B, EMB), jnp.float32),
    name="sc_gather",
    mesh=_sc_mesh,
    scratch_types=[
        pltpu.VMEM((_BG,), jnp.int32),
        pltpu.VMEM((_BG, EMB), jnp.float32),
        pltpu.SemaphoreType.DMA,
    ],
)
def _gather_kernel(emb_hbm, idx_hbm, out_hbm, idx_v, rows_v, sem):
    c = lax.axis_index("c")
    s = lax.axis_index("s")
    w = c * NS + s
    pltpu.sync_copy(idx_hbm.at[pl.ds(w * _BG, _BG)], idx_v)
    pltpu.async_copy(emb_hbm.at[idx_v], rows_v, sem).wait()
    pltpu.sync_copy(rows_v, out_hbm.at[pl.ds(w * _BG, _BG)])


_RB = 400


def _dis_from_degs(degs_ref):
    deg = degs_ref[0, :, 0:1] + degs_ref[1, :, 0:1] + 1.0
    return lax.rsqrt(deg)


def _tc_l1_body(x_ref, w1_ref, degs_ref, out_ref, dis_ref):
    dis = _dis_from_degs(degs_ref)
    h = jnp.dot(x_ref[...], w1_ref[...], preferred_element_type=jnp.float32)
    g = h * dis
    out_ref[0] = g[:, :EMB]
    out_ref[1] = g[:, EMB:]
    dis_ref[...] = jnp.broadcast_to(dis, (dis.shape[0], 8))


def _tc_l2_body(acc_ref, g_ref, dis8_ref, b1_ref, w2_ref, out_ref):
    dis = dis8_ref[:, 0:1]
    o0 = jnp.maximum(dis * (acc_ref[0] + g_ref[0]) + b1_ref[0, :EMB], 0.0)
    o1 = jnp.maximum(dis * (acc_ref[1] + g_ref[1]) + b1_ref[0, EMB:], 0.0)
    h2 = (jnp.dot(o0, w2_ref[:EMB, :], preferred_element_type=jnp.float32)
          + jnp.dot(o1, w2_ref[EMB:, :], preferred_element_type=jnp.float32))
    out_ref[...] = h2 * dis


def _tc_emb_body(acc_ref, g_ref, dis8_ref, b2_ref, out_ref):
    dis = dis8_ref[:, 0:1]
    out_ref[...] = dis * (acc_ref[0] + acc_ref[1] + g_ref[...]) + b2_ref[...]


_MB = 512


def _tc_mlp_body(x_ref, sk_ref, ow1_ref, ob1_ref, ow2_ref, ob2_ref,
                 mw1_ref, mb1_ref, mw2_ref, mb2_ref, mw3_ref, mb3_ref, out_ref):
    o = jnp.maximum(
        jnp.dot(x_ref[...], ow1_ref[...], preferred_element_type=jnp.float32)
        + ob1_ref[...], 0.0)
    occ = jnp.dot(o, ow2_ref[...], preferred_element_type=jnp.float32) + ob2_ref[...]
    z = (jnp.dot(occ, mw1_ref[:EMB, :], preferred_element_type=jnp.float32)
         + jnp.dot(sk_ref[...], mw1_ref[EMB:, :], preferred_element_type=jnp.float32)
         + mb1_ref[...])
    z = jnp.maximum(z, 0.0)
    z = jnp.maximum(
        jnp.dot(z, mw2_ref[...], preferred_element_type=jnp.float32) + mb2_ref[...],
        0.0)
    out_ref[...] = jax.nn.sigmoid(
        jnp.dot(z, mw3_ref[...], preferred_element_type=jnp.float32) + mb3_ref[...])


def _row_block(*dims):
    shape = dims
    nlead = len(dims) - 2

    def imap(i):
        return (0,) * nlead + (i,) + (0,)

    return pl.BlockSpec(shape, imap)


def _full(shape):
    return pl.BlockSpec(shape, lambda i: (0,) * len(shape))


def kernel(occupation_features, skill_idx, node_x, edge_index,
           gcn_w1, gcn_b1, gcn_w2, gcn_b2,
           occ_w1, occ_b1, occ_w2, occ_b2,
           mp_w1, mp_b1, mp_w2, mp_b2, mp_w3, mp_b3):
    src = edge_index[0].astype(jnp.int32)
    dst = edge_index[1].astype(jnp.int32)
    pad = E_PAD - E
    ar = jnp.arange(pad, dtype=jnp.int32)
    src_p = jnp.concatenate([src, (ar * 997) % N])
    dst_p = jnp.concatenate([dst, N + (ar % 16)])

    src3w = src_p.reshape(NW, PC_W, CHUNK)
    dst3w = dst_p.reshape(NW, PC_W, CHUNK)
    src3s = src_p.reshape(NS, 2, HALF, CHUNK)
    dst3s = dst_p.reshape(NS, 2, HALF, CHUNK)
    srcall = jnp.stack([src3s, src3s + N])

    ones128 = jnp.ones((CHUNK, 128), jnp.float32)
    zeros128 = jnp.zeros((N_ACC, EMB), jnp.float32)

    degs = _deg_kernel(dst3w, ones128, zeros128)

    gtab1, dis8 = pl.pallas_call(
        _tc_l1_body,
        grid=(N // _RB,),
        in_specs=[_row_block(_RB, D_IN), _full((D_IN, H)), _row_block(2, _RB, 128)],
        out_specs=[_row_block(2, _RB, EMB), _row_block(_RB, 8)],
        out_shape=[jax.ShapeDtypeStruct((2, N, EMB), jnp.float32),
                   jax.ShapeDtypeStruct((N, 8), jnp.float32)],
    )(node_x, gcn_w1, degs)

    acc1 = _msg1_kernel(gtab1.reshape(2 * N, EMB), srcall, dst3s, zeros128)

    g2 = pl.pallas_call(
        _tc_l2_body,
        grid=(N // _RB,),
        in_specs=[_row_block(2, _RB, EMB), _row_block(2, _RB, EMB),
                  _row_block(_RB, 8), _full((1, H)), _full((H, EMB))],
        out_specs=_row_block(_RB, EMB),
        out_shape=jax.ShapeDtypeStruct((N, EMB), jnp.float32),
    )(acc1, gtab1, dis8, gcn_b1.reshape(1, H), gcn_w2)

    acc2 = _msg2_kernel(g2, src3w, dst3w, zeros128)

    emb = pl.pallas_call(
        _tc_emb_body,
        grid=(N // _RB,),
        in_specs=[_row_block(2, _RB, EMB), _row_block(_RB, EMB),
                  _row_block(_RB, 8), _full((1, EMB))],
        out_specs=_row_block(_RB, EMB),
        out_shape=jax.ShapeDtypeStruct((N, EMB), jnp.float32),
    )(acc2, g2, dis8, gcn_b2.reshape(1, EMB))

    skill_emb = _gather_kernel(emb, skill_idx.astype(jnp.int32))

    scores = pl.pallas_call(
        _tc_mlp_body,
        grid=(B // _MB,),
        in_specs=[_row_block(_MB, D_IN), _row_block(_MB, EMB),
                  _full((D_IN, H)), _full((1, H)), _full((H, EMB)), _full((1, EMB)),
                  _full((2 * EMB, H)), _full((1, H)), _full((H, H // 2)),
                  _full((1, H // 2)), _full((H // 2, 1)), _full((1, 1))],
        out_specs=_row_block(_MB, 1),
        out_shape=jax.ShapeDtypeStruct((B, 1), jnp.float32),
    )(occupation_features, skill_emb,
      occ_w1, occ_b1.reshape(1, H), occ_w2, occ_b2.reshape(1, EMB),
      mp_w1, mp_b1.reshape(1, H), mp_w2, mp_b2.reshape(1, H // 2),
      mp_w3, mp_b3.reshape(1, 1))

    return scores.reshape(B)


# speedup vs baseline: 18.7318x; 1.1076x over previous
import functools

import jax
import jax.numpy as jnp
from jax import lax
from jax.experimental import pallas as pl
from jax.experimental.pallas import tpu as pltpu
from jax.experimental.pallas import tpu_sc as plsc

N = 10000
E = 160000
D_IN = 256
H = 256
EMB = 128
B = 4096

NC, NS = 2, 16
NW = NC * NS
CHUNK = 96
E_PAD = 165888
PC_W = E_PAD // NW // CHUNK
PC_S = E_PAD // NS // CHUNK
HALF = PC_S // 2
N_ACC = 10112
RZ = N_ACC // NS

_sc_mesh = plsc.VectorSubcoreMesh(core_axis_name="c", subcore_axis_name="s")


@functools.partial(
    pl.kernel,
    out_type=jax.ShapeDtypeStruct((NC, N_ACC, 128), jnp.float32),
    name="sc_deg",
    mesh=_sc_mesh,
    scratch_types=[
        pltpu.VMEM((PC_W, CHUNK), jnp.int32),
        pltpu.VMEM((CHUNK, 128), jnp.float32),
        pltpu.VMEM_SHARED((N_ACC, 128), jnp.float32),
        pltpu.SemaphoreType.DMA,
    ],
)
def _deg_kernel(dst_hbm, ones_hbm, zeros_hbm, deg_out, idx_d, ones_v, deg_sh, sem):
    c = lax.axis_index("c")
    s = lax.axis_index("s")
    pltpu.sync_copy(zeros_hbm.at[pl.ds(s * RZ, RZ)], deg_sh.at[pl.ds(s * RZ, RZ)])
    pltpu.sync_copy(ones_hbm, ones_v)
    pltpu.sync_copy(dst_hbm.at[c * NS + s], idx_d)
    plsc.subcore_barrier()

    def body(j, _):
        pltpu.sync_copy(ones_v, deg_sh.at[idx_d.at[j]], add=True)
        return 0

    lax.fori_loop(0, PC_W, body, 0)
    plsc.subcore_barrier()
    pltpu.sync_copy(deg_sh.at[pl.ds(s * RZ, RZ)], deg_out.at[c, pl.ds(s * RZ, RZ)])


def _pipelined_msg(gtab_hbm, idx_s, idx_d, rows0, rows1, acc_sh, sem0, sem1,
                   nchunks):
    pltpu.async_copy(gtab_hbm.at[idx_s.at[0]], rows0, sem0)

    def body(jj, _):
        j0 = 2 * jj
        j1 = j0 + 1
        pltpu.async_copy(gtab_hbm.at[idx_s.at[j1]], rows1, sem1)
        pltpu.make_async_copy(gtab_hbm.at[idx_s.at[j0]], rows0, sem0).wait()
        pltpu.sync_copy(rows0, acc_sh.at[idx_d.at[j0]], add=True)

        @pl.when(j0 + 2 < nchunks)
        def _():
            pltpu.async_copy(gtab_hbm.at[idx_s.at[j0 + 2]], rows0, sem0)

        pltpu.make_async_copy(gtab_hbm.at[idx_s.at[j1]], rows1, sem1).wait()
        pltpu.sync_copy(rows1, acc_sh.at[idx_d.at[j1]], add=True)
        return 0

    lax.fori_loop(0, nchunks // 2, body, 0)


@functools.partial(
    pl.kernel,
    out_type=jax.ShapeDtypeStruct((NC, N_ACC, EMB), jnp.float32),
    name="sc_msg1",
    mesh=_sc_mesh,
    scratch_types=[
        pltpu.VMEM((HALF, CHUNK), jnp.int32),
        pltpu.VMEM((HALF, CHUNK), jnp.int32),
        pltpu.VMEM((CHUNK, EMB), jnp.float32),
        pltpu.VMEM((CHUNK, EMB), jnp.float32),
        pltpu.VMEM_SHARED((N_ACC, EMB), jnp.float32),
        pltpu.SemaphoreType.DMA,
        pltpu.SemaphoreType.DMA,
    ],
)
def _msg1_kernel(gtab_hbm, src_hbm, dst_hbm, zeros_hbm, acc_out,
                 idx_s, idx_d, rows0, rows1, acc_sh, sem0, sem1):
    c = lax.axis_index("c")
    s = lax.axis_index("s")
    pltpu.sync_copy(zeros_hbm.at[pl.ds(s * RZ, RZ)], acc_sh.at[pl.ds(s * RZ, RZ)])
    plsc.subcore_barrier()
    for ph in range(2):
        pltpu.sync_copy(src_hbm.at[c, s, ph], idx_s)
        pltpu.sync_copy(dst_hbm.at[s, ph], idx_d)
        _pipelined_msg(gtab_hbm, idx_s, idx_d, rows0, rows1, acc_sh, sem0, sem1,
                       HALF)
    plsc.subcore_barrier()
    pltpu.sync_copy(acc_sh.at[pl.ds(s * RZ, RZ)], acc_out.at[c, pl.ds(s * RZ, RZ)])


@functools.partial(
    pl.kernel,
    out_type=jax.ShapeDtypeStruct((NC, N_ACC, EMB), jnp.float32),
    name="sc_msg2",
    mesh=_sc_mesh,
    scratch_types=[
        pltpu.VMEM((PC_W, CHUNK), jnp.int32),
        pltpu.VMEM((PC_W, CHUNK), jnp.int32),
        pltpu.VMEM((CHUNK, EMB), jnp.float32),
        pltpu.VMEM((CHUNK, EMB), jnp.float32),
        pltpu.VMEM_SHARED((N_ACC, EMB), jnp.float32),
        pltpu.SemaphoreType.DMA,
        pltpu.SemaphoreType.DMA,
    ],
)
def _msg2_kernel(gtab_hbm, src_hbm, dst_hbm, zeros_hbm, acc_out,
                 idx_s, idx_d, rows0, rows1, acc_sh, sem0, sem1):
    c = lax.axis_index("c")
    s = lax.axis_index("s")
    w = c * NS + s
    pltpu.sync_copy(zeros_hbm.at[pl.ds(s * RZ, RZ)], acc_sh.at[pl.ds(s * RZ, RZ)])
    pltpu.sync_copy(src_hbm.at[w], idx_s)
    pltpu.sync_copy(dst_hbm.at[w], idx_d)
    plsc.subcore_barrier()
    _pipelined_msg(gtab_hbm, idx_s, idx_d, rows0, rows1, acc_sh, sem0, sem1, PC_W)
    plsc.subcore_barrier()
    pltpu.sync_copy(acc_sh.at[pl.ds(s * RZ, RZ)], acc_out.at[c, pl.ds(s * RZ, RZ)])


_BG = B // NW


@functools.partial(
    pl.kernel,
    out_type=jax.ShapeDtypeStruct((B, EMB), jnp.float32),
    name="sc_gather",
    mesh=_sc_mesh,
    scratch_types=[
        pltpu.VMEM((_BG,), jnp.int32),
        pltpu.VMEM((_BG, EMB), jnp.float32),
        pltpu.SemaphoreType.DMA,
    ],
)
def _gather_kernel(emb_hbm, idx_hbm, out_hbm, idx_v, rows_v, sem):
    c = lax.axis_index("c")
    s = lax.axis_index("s")
    w = c * NS + s
    pltpu.sync_copy(idx_hbm.at[pl.ds(w * _BG, _BG)], idx_v)
    pltpu.async_copy(emb_hbm.at[idx_v], rows_v, sem).wait()
    pltpu.sync_copy(rows_v, out_hbm.at[pl.ds(w * _BG, _BG)])


_RB = 2000


def _dis_from_degs(degs_ref):
    deg = degs_ref[0, :, 0:1] + degs_ref[1, :, 0:1] + 1.0
    return lax.rsqrt(deg)


def _tc_l1_body(x_ref, w1_ref, degs_ref, out_ref, dis_ref):
    dis = _dis_from_degs(degs_ref)
    h = jnp.dot(x_ref[...], w1_ref[...], preferred_element_type=jnp.float32)
    g = h * dis
    out_ref[0] = g[:, :EMB]
    out_ref[1] = g[:, EMB:]
    dis_ref[...] = jnp.broadcast_to(dis, (dis.shape[0], 8))


def _tc_l2_body(acc_ref, g_ref, dis8_ref, b1_ref, w2_ref, out_ref):
    dis = dis8_ref[:, 0:1]
    o0 = jnp.maximum(dis * (acc_ref[0] + g_ref[0]) + b1_ref[0, :EMB], 0.0)
    o1 = jnp.maximum(dis * (acc_ref[1] + g_ref[1]) + b1_ref[0, EMB:], 0.0)
    h2 = (jnp.dot(o0, w2_ref[:EMB, :], preferred_element_type=jnp.float32)
          + jnp.dot(o1, w2_ref[EMB:, :], preferred_element_type=jnp.float32))
    out_ref[...] = h2 * dis


def _tc_emb_body(acc_ref, g_ref, dis8_ref, b2_ref, out_ref):
    dis = dis8_ref[:, 0:1]
    out_ref[...] = dis * (acc_ref[0] + acc_ref[1] + g_ref[...]) + b2_ref[...]


_MB = 2048


def _tc_mlp_body(x_ref, sk_ref, ow1_ref, ob1_ref, ow2_ref, ob2_ref,
                 mw1_ref, mb1_ref, mw2_ref, mb2_ref, mw3_ref, mb3_ref, out_ref):
    o = jnp.maximum(
        jnp.dot(x_ref[...], ow1_ref[...], preferred_element_type=jnp.float32)
        + ob1_ref[...], 0.0)
    occ = jnp.dot(o, ow2_ref[...], preferred_element_type=jnp.float32) + ob2_ref[...]
    z = (jnp.dot(occ, mw1_ref[:EMB, :], preferred_element_type=jnp.float32)
         + jnp.dot(sk_ref[...], mw1_ref[EMB:, :], preferred_element_type=jnp.float32)
         + mb1_ref[...])
    z = jnp.maximum(z, 0.0)
    z = jnp.maximum(
        jnp.dot(z, mw2_ref[...], preferred_element_type=jnp.float32) + mb2_ref[...],
        0.0)
    out_ref[...] = jax.nn.sigmoid(
        jnp.dot(z, mw3_ref[...], preferred_element_type=jnp.float32) + mb3_ref[...])


def _row_block(*dims):
    shape = dims
    nlead = len(dims) - 2

    def imap(i):
        return (0,) * nlead + (i,) + (0,)

    return pl.BlockSpec(shape, imap)


def _full(shape):
    return pl.BlockSpec(shape, lambda i: (0,) * len(shape))


def kernel(occupation_features, skill_idx, node_x, edge_index,
           gcn_w1, gcn_b1, gcn_w2, gcn_b2,
           occ_w1, occ_b1, occ_w2, occ_b2,
           mp_w1, mp_b1, mp_w2, mp_b2, mp_w3, mp_b3):
    src = edge_index[0].astype(jnp.int32)
    dst = edge_index[1].astype(jnp.int32)
    pad = E_PAD - E
    ar = jnp.arange(pad, dtype=jnp.int32)
    src_p = jnp.concatenate([src, (ar * 997) % N])
    dst_p = jnp.concatenate([dst, N + (ar % 16)])

    src3w = src_p.reshape(NW, PC_W, CHUNK)
    dst3w = dst_p.reshape(NW, PC_W, CHUNK)
    src3s = src_p.reshape(NS, 2, HALF, CHUNK)
    dst3s = dst_p.reshape(NS, 2, HALF, CHUNK)
    srcall = jnp.stack([src3s, src3s + N])

    ones128 = jnp.ones((CHUNK, 128), jnp.float32)
    zeros128 = jnp.zeros((N_ACC, EMB), jnp.float32)

    degs = _deg_kernel(dst3w, ones128, zeros128)
    degs8 = lax.slice(degs, (0, 0, 0), (2, N_ACC, 8))

    gtab1, dis8 = pl.pallas_call(
        _tc_l1_body,
        grid=(N // _RB,),
        in_specs=[_row_block(_RB, D_IN), _full((D_IN, H)), _row_block(2, _RB, 8)],
        out_specs=[_row_block(2, _RB, EMB), _row_block(_RB, 8)],
        out_shape=[jax.ShapeDtypeStruct((2, N, EMB), jnp.float32),
                   jax.ShapeDtypeStruct((N, 8), jnp.float32)],
    )(node_x, gcn_w1, degs8)

    acc1 = _msg1_kernel(gtab1.reshape(2 * N, EMB), srcall, dst3s, zeros128)

    g2 = pl.pallas_call(
        _tc_l2_body,
        grid=(N // _RB,),
        in_specs=[_row_block(2, _RB, EMB), _row_block(2, _RB, EMB),
                  _row_block(_RB, 8), _full((1, H)), _full((H, EMB))],
        out_specs=_row_block(_RB, EMB),
        out_shape=jax.ShapeDtypeStruct((N, EMB), jnp.float32),
    )(acc1, gtab1, dis8, gcn_b1.reshape(1, H), gcn_w2)

    acc2 = _msg2_kernel(g2, src3w, dst3w, zeros128)

    emb = pl.pallas_call(
        _tc_emb_body,
        grid=(N // _RB,),
        in_specs=[_row_block(2, _RB, EMB), _row_block(_RB, EMB),
                  _row_block(_RB, 8), _full((1, EMB))],
        out_specs=_row_block(_RB, EMB),
        out_shape=jax.ShapeDtypeStruct((N, EMB), jnp.float32),
    )(acc2, g2, dis8, gcn_b2.reshape(1, EMB))

    skill_emb = _gather_kernel(emb, skill_idx.astype(jnp.int32))

    scores = pl.pallas_call(
        _tc_mlp_body,
        grid=(B // _MB,),
        in_specs=[_row_block(_MB, D_IN), _row_block(_MB, EMB),
                  _full((D_IN, H)), _full((1, H)), _full((H, EMB)), _full((1, EMB)),
                  _full((2 * EMB, H)), _full((1, H)), _full((H, H // 2)),
                  _full((1, H // 2)), _full((H // 2, 1)), _full((1, 1))],
        out_specs=_row_block(_MB, 1),
        out_shape=jax.ShapeDtypeStruct((B, 1), jnp.float32),
    )(occupation_features, skill_emb,
      occ_w1, occ_b1.reshape(1, H), occ_w2, occ_b2.reshape(1, EMB),
      mp_w1, mp_b1.reshape(1, H), mp_w2, mp_b2.reshape(1, H // 2),
      mp_w3, mp_b3.reshape(1, 1))

    return scores.reshape(B)
